# Initial kernel scaffold; baseline (speedup 1.0000x reference)
import jax
import jax.numpy as jnp
from jax.experimental import pallas as pl


def kernel(x, edge_index, edge_weight, W1, b1, W2, b2, Wd1, bd1, Wd2, bd2):
    raise NotImplementedError("write your pallas kernel here")


# speedup vs baseline: 7.6175x; 7.6175x over previous
import functools

import jax
import jax.numpy as jnp
import numpy as np
from jax import lax
from jax.experimental import pallas as pl
from jax.experimental.pallas import tpu as pltpu
from jax.experimental.pallas import tpu_sc as plsc

_N = 10000
_E = 320000
_DF = 128
_LAT = 7
_HP = 16
_SP_INV1 = float(np.log(np.expm1(1.0)))

_CHUNK = 128
_NCHUNKS = _E // _CHUNK
_NC = 2
_NS = 16
_NW = _NC * _NS
_CPW_LO = _NCHUNKS // _NW
_CPW_REM = _NCHUNKS - _CPW_LO * _NW
_NP = 10112
_RPT = _NP // _NS

_ROWS_BLK = 2000
_GRID = _N // _ROWS_BLK

_sc_mesh = plsc.VectorSubcoreMesh(core_axis_name="c", subcore_axis_name="s")


@functools.partial(
    pl.kernel,
    out_type=jax.ShapeDtypeStruct((_NC, _NP, _HP), jnp.float32),
    mesh=_sc_mesh,
    scratch_types=[
        pltpu.VMEM((_CHUNK,), jnp.int32),
        pltpu.VMEM((_CHUNK,), jnp.int32),
        pltpu.VMEM((_CHUNK,), jnp.float32),
        pltpu.VMEM((_CHUNK, _HP), jnp.float32),
        pltpu.VMEM((_RPT, _HP), jnp.float32),
        pltpu.VMEM_SHARED((_NP, _HP), jnp.float32),
        pltpu.SemaphoreType.DMA,
    ],
    compiler_params=pltpu.CompilerParams(use_tc_tiling_on_sc=False),
)
def _spmm_sc(h_hbm, ei_hbm, w_hbm, out_hbm, rvm, cvm, wvm, gvm, zvm, acc, sem):
    c = lax.axis_index("c")
    s = lax.axis_index("s")
    wid = c * _NS + s

    zero_row = jnp.zeros((_HP,), jnp.float32)

    def _zero(i, carry):
        zvm[i, :] = zero_row
        return carry

    lax.fori_loop(0, _RPT, _zero, 0)
    pltpu.sync_copy(zvm, acc.at[pl.ds(s * _RPT, _RPT)])
    plsc.subcore_barrier()

    start = wid * _CPW_LO + jnp.minimum(wid, _CPW_REM)
    count = _CPW_LO + jnp.where(wid < _CPW_REM, 1, 0)

    def _chunk(j, carry):
        base = pl.multiple_of((start + j) * _CHUNK, _CHUNK)
        pltpu.sync_copy(ei_hbm.at[0, pl.ds(base, _CHUNK)], rvm)
        pltpu.sync_copy(ei_hbm.at[1, pl.ds(base, _CHUNK)], cvm)
        pltpu.sync_copy(w_hbm.at[pl.ds(base, _CHUNK)], wvm)
        pltpu.async_copy(h_hbm.at[cvm], gvm, sem).wait()

        def _scale(g, inner):
            base16 = pl.multiple_of(g * 16, 16)
            wv = wvm[pl.ds(base16, 16)]
            for e in range(16):
                gvm[base16 + e, :] = gvm[base16 + e, :] * wv[e]
            return inner

        lax.fori_loop(0, _CHUNK // 16, _scale, 0)
        pltpu.sync_copy(gvm, acc.at[rvm], add=True)
        return carry

    lax.fori_loop(0, count, _chunk, 0)
    plsc.subcore_barrier()
    pltpu.sync_copy(acc.at[pl.ds(s * _RPT, _RPT)],
                    out_hbm.at[c, pl.ds(s * _RPT, _RPT)])


def _mm_body(x_ref, w_ref, o_ref):
    o_ref[...] = jnp.dot(x_ref[...], w_ref[...],
                         preferred_element_type=jnp.float32)


_mm = pl.pallas_call(
    _mm_body,
    grid=(_GRID,),
    in_specs=[
        pl.BlockSpec((_ROWS_BLK, _DF), lambda i: (i, 0)),
        pl.BlockSpec((_DF, _HP), lambda i: (0, 0)),
    ],
    out_specs=pl.BlockSpec((_ROWS_BLK, _HP), lambda i: (i, 0)),
    out_shape=jax.ShapeDtypeStruct((_N, _HP), jnp.float32),
)


def _layer2_body(parts_ref, b_ref, w_ref, o_ref):
    lat = jnp.maximum(parts_ref[0] + parts_ref[1] + b_ref[0], 0.0)
    o_ref[...] = jnp.dot(lat, w_ref[...], preferred_element_type=jnp.float32)


_layer2 = pl.pallas_call(
    _layer2_body,
    grid=(_GRID,),
    in_specs=[
        pl.BlockSpec((_NC, _ROWS_BLK, _HP), lambda i: (0, i, 0)),
        pl.BlockSpec((1, _HP), lambda i: (0, 0)),
        pl.BlockSpec((_HP, _HP), lambda i: (0, 0)),
    ],
    out_specs=pl.BlockSpec((_ROWS_BLK, _HP), lambda i: (i, 0)),
    out_shape=jax.ShapeDtypeStruct((_N, _HP), jnp.float32),
)


def _head_body(parts_ref, b2_ref, wd1_ref, bd1_ref, wd2_ref, bd2_ref, o_ref):
    lat = jnp.maximum(parts_ref[0] + parts_ref[1] + b2_ref[0], 0.0)
    pd = jnp.tanh(jnp.dot(lat, wd1_ref[...],
                          preferred_element_type=jnp.float32) + bd1_ref[0])
    pp = jnp.tanh(jnp.dot(lat, wd2_ref[...],
                          preferred_element_type=jnp.float32) + bd2_ref[0])
    colid = lax.broadcasted_iota(jnp.int32, pd.shape, 1)
    head = jnp.where(colid < _LAT, pd, jax.nn.softplus(pd + _SP_INV1))
    o_ref[...] = jnp.concatenate([head[:, : 2 * _LAT], pp], axis=1)


_head = pl.pallas_call(
    _head_body,
    grid=(_GRID,),
    in_specs=[
        pl.BlockSpec((_NC, _ROWS_BLK, _HP), lambda i: (0, i, 0)),
        pl.BlockSpec((1, _HP), lambda i: (0, 0)),
        pl.BlockSpec((_HP, _HP), lambda i: (0, 0)),
        pl.BlockSpec((1, _HP), lambda i: (0, 0)),
        pl.BlockSpec((_HP, _HP), lambda i: (0, 0)),
        pl.BlockSpec((1, _HP), lambda i: (0, 0)),
    ],
    out_specs=pl.BlockSpec((_ROWS_BLK, 2 * _LAT + _HP), lambda i: (i, 0)),
    out_shape=jax.ShapeDtypeStruct((_N, 2 * _LAT + _HP), jnp.float32),
)


def _pad_w(w, rows, cols):
    return jnp.pad(w, ((0, rows - w.shape[0]), (0, cols - w.shape[1])))


def _pad_b(b):
    return jnp.pad(b, (0, _HP - b.shape[0])).reshape(1, _HP)


def kernel(x, edge_index, edge_weight, W1, b1, W2, b2, Wd1, bd1, Wd2, bd2):
    ei = edge_index.astype(jnp.int32)
    W1p = _pad_w(W1, _DF, _HP)
    W2p = _pad_w(W2, _HP, _HP)
    Wd1p = _pad_w(Wd1, _HP, _HP)
    Wd2p = _pad_w(Wd2, _HP, _HP)

    h1 = _mm(x, W1p)
    s1 = _spmm_sc(h1, ei, edge_weight)
    h2 = _layer2(s1, _pad_b(b1), W2p)
    s2 = _spmm_sc(h2, ei, edge_weight)
    return _head(s2, _pad_b(b2), Wd1p, _pad_b(bd1), Wd2p, _pad_b(bd2))
